# baseline (speedup 1.0000x reference)
import functools

import jax
import jax.numpy as jnp
from jax import lax
from jax.experimental import pallas as pl
from jax.experimental.pallas import tpu as pltpu
from jax.experimental.pallas import tpu_sc as plsc

N_NODES = 10000
D = 128

NPAD = 10240
NBLK = NPAD // 128
NW = 32
B = 128
NCH = 80
EW = NCH * B
EPAD = NW * EW
ROWS_PER_TILE = NPAD // 16

_MESH = dict(core_axis_name="c", subcore_axis_name="s")


def _sc_degree(dst_flat):
  @functools.partial(
      pl.kernel,
      mesh=plsc.VectorSubcoreMesh(**_MESH),
      out_type=jax.ShapeDtypeStruct((2 * NPAD, D), jnp.float32),
      scratch_types=[
          pltpu.VMEM((B,), jnp.int32),
          pltpu.VMEM((B,), jnp.int32),
          pltpu.VMEM((B, D), jnp.float32),
          pltpu.VMEM_SHARED((NPAD, D), jnp.float32),
          pltpu.SemaphoreType.DMA,
          pltpu.SemaphoreType.DMA,
      ],
  )
  def k(dst_hbm, out_hbm, cidx0, cidx1, ones_v, acc, t0, t1):
    c = lax.axis_index("c")
    s = lax.axis_index("s")
    w = c * 16 + s

    def fill_zero(t, _):
      i = t // 8
      j = lax.rem(t, 8) * 16
      ones_v[i, pl.ds(j, 16)] = jnp.zeros((16,), jnp.float32)
      return 0
    lax.fori_loop(0, B * 8, fill_zero, 0)
    for t in range(ROWS_PER_TILE // B):
      pltpu.sync_copy(ones_v, acc.at[pl.ds(s * ROWS_PER_TILE + t * B, B)])
    plsc.subcore_barrier()

    def fill_ones(t, _):
      i = t // 8
      j = lax.rem(t, 8) * 16
      ones_v[i, pl.ds(j, 16)] = jnp.full((16,), 1.0, jnp.float32)
      return 0
    lax.fori_loop(0, B * 8, fill_ones, 0)

    base = w * EW

    pltpu.sync_copy(dst_hbm.at[pl.ds(base, B)], cidx0)
    pltpu.async_copy(ones_v, acc.at[cidx0], t0, add=True)
    pltpu.sync_copy(dst_hbm.at[pl.ds(base + B, B)], cidx1)
    pltpu.async_copy(ones_v, acc.at[cidx1], t1, add=True)

    def body(g, _):
      j0 = 2 * g
      pltpu.make_async_copy(ones_v, acc.at[cidx0], t0).wait()
      pltpu.sync_copy(dst_hbm.at[pl.ds(base + (j0 + 2) * B, B)], cidx0)
      pltpu.async_copy(ones_v, acc.at[cidx0], t0, add=True)
      pltpu.make_async_copy(ones_v, acc.at[cidx1], t1).wait()
      pltpu.sync_copy(dst_hbm.at[pl.ds(base + (j0 + 3) * B, B)], cidx1)
      pltpu.async_copy(ones_v, acc.at[cidx1], t1, add=True)
      return 0
    lax.fori_loop(0, NCH // 2 - 1, body, 0)

    pltpu.make_async_copy(ones_v, acc.at[cidx0], t0).wait()
    pltpu.make_async_copy(ones_v, acc.at[cidx1], t1).wait()

    plsc.subcore_barrier()
    for t in range(ROWS_PER_TILE // B):
      r = s * ROWS_PER_TILE + t * B
      pltpu.sync_copy(acc.at[pl.ds(r, B)], ones_v)
      pltpu.sync_copy(ones_v, out_hbm.at[pl.ds(c * NPAD + r, B)])

  return k(dst_flat)


def _sc_aggregate(hs, src_flat, dst_flat):
  @functools.partial(
      pl.kernel,
      mesh=plsc.VectorSubcoreMesh(**_MESH),
      out_type=jax.ShapeDtypeStruct((2 * NPAD, D), jnp.float32),
      scratch_types=(
          [pltpu.VMEM((B,), jnp.int32)] * 2 +
          [pltpu.VMEM((B,), jnp.int32)] * 2 +
          [pltpu.VMEM((B, D), jnp.float32)] * 2 +
          [pltpu.VMEM_SHARED((NPAD, D), jnp.float32)] +
          [pltpu.SemaphoreType.DMA] * 4
      ),
  )
  def k(hs_hbm, src_hbm, dst_hbm, out_hbm, *sc):
    sidx = sc[0:2]
    didx = sc[2:4]
    buf = sc[4:6]
    acc = sc[6]
    gsem = sc[7:9]
    tsem = sc[9:11]
    c = lax.axis_index("c")
    s = lax.axis_index("s")
    w = c * 16 + s

    def fill_zero(t, _):
      i = t // 8
      j = lax.rem(t, 8) * 16
      buf[0][i, pl.ds(j, 16)] = jnp.zeros((16,), jnp.float32)
      return 0
    lax.fori_loop(0, B * 8, fill_zero, 0)

    for t in range(ROWS_PER_TILE // B):
      pltpu.sync_copy(buf[0], acc.at[pl.ds(s * ROWS_PER_TILE + t * B, B)])
    plsc.subcore_barrier()

    base = w * EW

    for b in range(2):
      pltpu.sync_copy(src_hbm.at[pl.ds(base + b * B, B)], sidx[b])
      pltpu.sync_copy(dst_hbm.at[pl.ds(base + b * B, B)], didx[b])
      pltpu.async_copy(hs_hbm.at[sidx[b]], buf[b], gsem[b])

    def body(g, _):
      j = 2 * g
      for b in range(2):
        pltpu.make_async_copy(hs_hbm.at[sidx[b]], buf[b], gsem[b]).wait()
        pltpu.async_copy(buf[b], acc.at[didx[b]], tsem[b], add=True)
      for b in range(2):
        pltpu.make_async_copy(buf[b], acc.at[didx[b]], tsem[b]).wait()
        pltpu.sync_copy(src_hbm.at[pl.ds(base + (j + 2 + b) * B, B)], sidx[b])
        pltpu.sync_copy(dst_hbm.at[pl.ds(base + (j + 2 + b) * B, B)], didx[b])
        pltpu.async_copy(hs_hbm.at[sidx[b]], buf[b], gsem[b])
      return 0
    lax.fori_loop(0, NCH // 2 - 1, body, 0)

    for b in range(2):
      pltpu.make_async_copy(hs_hbm.at[sidx[b]], buf[b], gsem[b]).wait()
      pltpu.async_copy(buf[b], acc.at[didx[b]], tsem[b], add=True)
    for b in range(2):
      pltpu.make_async_copy(buf[b], acc.at[didx[b]], tsem[b]).wait()

    plsc.subcore_barrier()
    for t in range(ROWS_PER_TILE // B):
      r = s * ROWS_PER_TILE + t * B
      pltpu.sync_copy(acc.at[pl.ds(r, B)], buf[0])
      pltpu.sync_copy(buf[0], out_hbm.at[pl.ds(c * NPAD + r, B)])

  return k(hs, src_flat, dst_flat)


def _tc_prescale(x_pad, degp, w1):
  def body(x_ref, dp_ref, w_ref, hs_ref, dis_ref):
    deg = dp_ref[0, :, 0] + dp_ref[1, :, 0] + 1.0
    dis = lax.rsqrt(deg)
    h = jnp.dot(x_ref[...], w_ref[...], preferred_element_type=jnp.float32)
    hs_ref[...] = h * dis[:, None]
    dis_ref[...] = dis[:, None]

  return pl.pallas_call(
      body,
      grid=(NBLK,),
      in_specs=[
          pl.BlockSpec((128, D), lambda i: (i, 0)),
          pl.BlockSpec((2, 128, D), lambda i: (0, i, 0)),
          pl.BlockSpec((D, D), lambda i: (0, 0)),
      ],
      out_specs=[
          pl.BlockSpec((128, D), lambda i: (i, 0)),
          pl.BlockSpec((128, 1), lambda i: (i, 0)),
      ],
      out_shape=[
          jax.ShapeDtypeStruct((NPAD, D), jnp.float32),
          jax.ShapeDtypeStruct((NPAD, 1), jnp.float32),
      ],
  )(x_pad, degp, w1)


def _tc_mid(parts, hs1, dis, b1, w2):
  def body(p_ref, hs_ref, dis_ref, b_ref, w_ref, out_ref):
    agg = p_ref[0] + p_ref[1] + hs_ref[...]
    h = jnp.maximum(agg * dis_ref[...] + b_ref[...], 0.0)
    out_ref[...] = jnp.dot(
        h, w_ref[...], preferred_element_type=jnp.float32) * dis_ref[...]

  return pl.pallas_call(
      body,
      grid=(NBLK,),
      in_specs=[
          pl.BlockSpec((2, 128, D), lambda i: (0, i, 0)),
          pl.BlockSpec((128, D), lambda i: (i, 0)),
          pl.BlockSpec((128, 1), lambda i: (i, 0)),
          pl.BlockSpec((1, D), lambda i: (0, 0)),
          pl.BlockSpec((D, D), lambda i: (0, 0)),
      ],
      out_specs=pl.BlockSpec((128, D), lambda i: (i, 0)),
      out_shape=jax.ShapeDtypeStruct((NPAD, D), jnp.float32),
  )(parts, hs1, dis, b1, w2)


def _tc_final(parts, hs2, dis, b2):
  def body(p_ref, hs_ref, dis_ref, b_ref, out_ref):
    agg = p_ref[0] + p_ref[1] + hs_ref[...]
    out_ref[...] = agg * dis_ref[...] + b_ref[...]

  return pl.pallas_call(
      body,
      grid=(NBLK,),
      in_specs=[
          pl.BlockSpec((2, 128, D), lambda i: (0, i, 0)),
          pl.BlockSpec((128, D), lambda i: (i, 0)),
          pl.BlockSpec((128, 1), lambda i: (i, 0)),
          pl.BlockSpec((1, D), lambda i: (0, 0)),
      ],
      out_specs=pl.BlockSpec((128, D), lambda i: (i, 0)),
      out_shape=jax.ShapeDtypeStruct((NPAD, D), jnp.float32),
  )(parts, hs2, dis, b2)


def kernel(x, edge_index, W1, b1, W2, b2):
  src = edge_index[0].astype(jnp.int32)
  dst = edge_index[1].astype(jnp.int32)
  npad_e = EPAD - src.shape[0]
  src_flat = jnp.concatenate([src, jnp.zeros((npad_e,), jnp.int32)])
  dst_flat = jnp.concatenate([dst, jnp.full((npad_e,), N_NODES, jnp.int32)])
  x_pad = jnp.pad(x, ((0, NPAD - N_NODES), (0, 0)))
  b1r = b1.reshape(1, D)
  b2r = b2.reshape(1, D)

  degp = _sc_degree(dst_flat).reshape(2, NPAD, D)
  hs1, dis = _tc_prescale(x_pad, degp, W1)
  parts1 = _sc_aggregate(hs1, src_flat, dst_flat).reshape(2, NPAD, D)
  hs2 = _tc_mid(parts1, hs1, dis, b1r, W2)
  parts2 = _sc_aggregate(hs2, src_flat, dst_flat).reshape(2, NPAD, D)
  out = _tc_final(parts2, hs2, dis, b2r)
  return out[:N_NODES]


# speedup vs baseline: 1.1049x; 1.1049x over previous
import functools

import jax
import jax.numpy as jnp
from jax import lax
from jax.experimental import pallas as pl
from jax.experimental.pallas import tpu as pltpu
from jax.experimental.pallas import tpu_sc as plsc

N_NODES = 10000
D = 128

NPAD = 10240
NBLK = NPAD // 128
NW = 32
B = 128
NCH = 80
EW = NCH * B
EPAD = NW * EW
ROWS_PER_TILE = NPAD // 16
NT = ROWS_PER_TILE // B

_MESH = dict(core_axis_name="c", subcore_axis_name="s")


def _sc_degree(dst_flat, zer, one):
  @functools.partial(
      pl.kernel,
      mesh=plsc.VectorSubcoreMesh(**_MESH),
      out_type=jax.ShapeDtypeStruct((2 * NPAD, D), jnp.float32),
      scratch_types=(
          [pltpu.VMEM((B,), jnp.int32)] * 4 +
          [pltpu.VMEM((B, D), jnp.float32)] * 2 +
          [pltpu.VMEM_SHARED((NPAD, D), jnp.float32)] +
          [pltpu.SemaphoreType.DMA] * 8
      ),
  )
  def k(dst_hbm, zer_hbm, one_hbm, out_hbm, *sc):
    didx = sc[0:4]
    ones_v, zbuf = sc[4:6]
    acc = sc[6]
    jsem = sc[7:11]
    tsem = sc[11:13]
    wsem = sc[13:15]
    c = lax.axis_index("c")
    s = lax.axis_index("s")
    w = c * 16 + s
    base = w * EW

    for q in range(4):
      pltpu.async_copy(dst_hbm.at[pl.ds(base + q * B, B)], didx[q], jsem[q])

    pltpu.sync_copy(zer_hbm, zbuf)
    for t in range(NT):
      pltpu.sync_copy(zbuf, acc.at[pl.ds(s * ROWS_PER_TILE + t * B, B)])
    pltpu.sync_copy(one_hbm, ones_v)
    plsc.subcore_barrier()

    for b in range(2):
      pltpu.make_async_copy(
          dst_hbm.at[pl.ds(base + b * B, B)], didx[b], jsem[b]).wait()
      pltpu.async_copy(ones_v, acc.at[didx[b]], tsem[b], add=True)

    def body(g, _):
      j0 = 4 * g + 2
      for p in range(4):
        u = (p + 2) % 4
        b = p % 2
        u2 = p
        pltpu.make_async_copy(ones_v, acc.at[didx[u2]], tsem[b]).wait()
        pltpu.async_copy(
            dst_hbm.at[pl.ds(base + (j0 + p + 2) * B, B)], didx[u2], jsem[u2])
        pltpu.make_async_copy(
            dst_hbm.at[pl.ds(base + (j0 + p) * B, B)], didx[u], jsem[u]).wait()
        pltpu.async_copy(ones_v, acc.at[didx[u]], tsem[b], add=True)
      return 0
    lax.fori_loop(0, (NCH - 4) // 4, body, 0)

    for p in range(2):
      u = 2 + p
      pltpu.make_async_copy(ones_v, acc.at[didx[p]], tsem[p]).wait()
      pltpu.make_async_copy(
          dst_hbm.at[pl.ds(base + (NCH - 2 + p) * B, B)], didx[u], jsem[u]).wait()
      pltpu.async_copy(ones_v, acc.at[didx[u]], tsem[p], add=True)
    for p in range(2):
      pltpu.make_async_copy(ones_v, acc.at[didx[2 + p]], tsem[p]).wait()

    plsc.subcore_barrier()
    wb = (ones_v, zbuf)
    for t in range(NT):
      b = t % 2
      if t >= 2:
        rp = s * ROWS_PER_TILE + (t - 2) * B
        pltpu.make_async_copy(
            wb[b], out_hbm.at[pl.ds(c * NPAD + rp, B)], wsem[b]).wait()
      r = s * ROWS_PER_TILE + t * B
      pltpu.sync_copy(acc.at[pl.ds(r, B)], wb[b])
      pltpu.async_copy(wb[b], out_hbm.at[pl.ds(c * NPAD + r, B)], wsem[b])
    for t in range(NT - 2, NT):
      b = t % 2
      r = s * ROWS_PER_TILE + t * B
      pltpu.make_async_copy(
          wb[b], out_hbm.at[pl.ds(c * NPAD + r, B)], wsem[b]).wait()

  return k(dst_flat, zer, one)


def _sc_aggregate(hs, src_flat, dst_flat, zer):
  @functools.partial(
      pl.kernel,
      mesh=plsc.VectorSubcoreMesh(**_MESH),
      out_type=jax.ShapeDtypeStruct((2 * NPAD, D), jnp.float32),
      scratch_types=(
          [pltpu.VMEM((B,), jnp.int32)] * 4 +
          [pltpu.VMEM((B,), jnp.int32)] * 4 +
          [pltpu.VMEM((B, D), jnp.float32)] * 2 +
          [pltpu.VMEM_SHARED((NPAD, D), jnp.float32)] +
          [pltpu.SemaphoreType.DMA] * 12
      ),
  )
  def k(hs_hbm, src_hbm, dst_hbm, zer_hbm, out_hbm, *sc):
    sidx = sc[0:4]
    didx = sc[4:8]
    buf = sc[8:10]
    acc = sc[10]
    isem = sc[11:15]
    jsem = sc[15:19]
    gsem = sc[19:21]
    tsem = sc[21:23]
    c = lax.axis_index("c")
    s = lax.axis_index("s")
    w = c * 16 + s
    base = w * EW

    for q in range(4):
      pltpu.async_copy(src_hbm.at[pl.ds(base + q * B, B)], sidx[q], isem[q])
      pltpu.async_copy(dst_hbm.at[pl.ds(base + q * B, B)], didx[q], jsem[q])

    pltpu.sync_copy(zer_hbm, buf[0])
    for t in range(NT):
      pltpu.sync_copy(buf[0], acc.at[pl.ds(s * ROWS_PER_TILE + t * B, B)])

    for b in range(2):
      pltpu.make_async_copy(
          src_hbm.at[pl.ds(base + b * B, B)], sidx[b], isem[b]).wait()
      pltpu.make_async_copy(
          dst_hbm.at[pl.ds(base + b * B, B)], didx[b], jsem[b]).wait()
      pltpu.async_copy(hs_hbm.at[sidx[b]], buf[b], gsem[b])
    plsc.subcore_barrier()

    def body(g, _):
      j0 = 4 * g
      for p in range(2):
        for b in range(2):
          u = 2 * p + b
          pltpu.make_async_copy(hs_hbm.at[sidx[u]], buf[b], gsem[b]).wait()
          pltpu.async_copy(buf[b], acc.at[didx[u]], tsem[b], add=True)
        for b in range(2):
          u = 2 * p + b
          u2 = (u + 2) % 4
          off2 = base + (j0 + u + 2) * B
          off4 = base + (j0 + u + 4) * B
          pltpu.make_async_copy(buf[b], acc.at[didx[u]], tsem[b]).wait()
          pltpu.make_async_copy(
              src_hbm.at[pl.ds(off2, B)], sidx[u2], isem[u2]).wait()
          pltpu.make_async_copy(
              dst_hbm.at[pl.ds(off2, B)], didx[u2], jsem[u2]).wait()
          pltpu.async_copy(hs_hbm.at[sidx[u2]], buf[b], gsem[b])
          pltpu.async_copy(src_hbm.at[pl.ds(off4, B)], sidx[u], isem[u])
          pltpu.async_copy(dst_hbm.at[pl.ds(off4, B)], didx[u], jsem[u])
      return 0
    lax.fori_loop(0, (NCH - 4) // 4, body, 0)

    for b in range(2):
      pltpu.make_async_copy(hs_hbm.at[sidx[b]], buf[b], gsem[b]).wait()
      pltpu.async_copy(buf[b], acc.at[didx[b]], tsem[b], add=True)
    for b in range(2):
      u2 = b + 2
      off2 = base + (NCH - 2 + b) * B
      pltpu.make_async_copy(buf[b], acc.at[didx[b]], tsem[b]).wait()
      pltpu.make_async_copy(
          src_hbm.at[pl.ds(off2, B)], sidx[u2], isem[u2]).wait()
      pltpu.make_async_copy(
          dst_hbm.at[pl.ds(off2, B)], didx[u2], jsem[u2]).wait()
      pltpu.async_copy(hs_hbm.at[sidx[u2]], buf[b], gsem[b])
    for b in range(2):
      u = b + 2
      pltpu.make_async_copy(hs_hbm.at[sidx[u]], buf[b], gsem[b]).wait()
      pltpu.async_copy(buf[b], acc.at[didx[u]], tsem[b], add=True)
    for b in range(2):
      u = b + 2
      pltpu.make_async_copy(buf[b], acc.at[didx[u]], tsem[b]).wait()

    plsc.subcore_barrier()
    for t in range(NT):
      b = t % 2
      if t >= 2:
        rp = s * ROWS_PER_TILE + (t - 2) * B
        pltpu.make_async_copy(
            buf[b], out_hbm.at[pl.ds(c * NPAD + rp, B)], gsem[b]).wait()
      r = s * ROWS_PER_TILE + t * B
      pltpu.sync_copy(acc.at[pl.ds(r, B)], buf[b])
      pltpu.async_copy(buf[b], out_hbm.at[pl.ds(c * NPAD + r, B)], gsem[b])
    for t in range(NT - 2, NT):
      b = t % 2
      r = s * ROWS_PER_TILE + t * B
      pltpu.make_async_copy(
          buf[b], out_hbm.at[pl.ds(c * NPAD + r, B)], gsem[b]).wait()

  return k(hs, src_flat, dst_flat, zer)


def _tc_prescale(x_pad, degp, w1):
  def body(x_ref, dp_ref, w_ref, hs_ref, dis_ref):
    deg = dp_ref[0, :, 0] + dp_ref[1, :, 0] + 1.0
    dis = lax.rsqrt(deg)
    h = jnp.dot(x_ref[...], w_ref[...], preferred_element_type=jnp.float32)
    hs_ref[...] = h * dis[:, None]
    dis_ref[...] = dis[:, None]

  return pl.pallas_call(
      body,
      grid=(NBLK,),
      in_specs=[
          pl.BlockSpec((128, D), lambda i: (i, 0)),
          pl.BlockSpec((2, 128, D), lambda i: (0, i, 0)),
          pl.BlockSpec((D, D), lambda i: (0, 0)),
      ],
      out_specs=[
          pl.BlockSpec((128, D), lambda i: (i, 0)),
          pl.BlockSpec((128, 1), lambda i: (i, 0)),
      ],
      out_shape=[
          jax.ShapeDtypeStruct((NPAD, D), jnp.float32),
          jax.ShapeDtypeStruct((NPAD, 1), jnp.float32),
      ],
  )(x_pad, degp, w1)


def _tc_mid(parts, hs1, dis, b1, w2):
  def body(p_ref, hs_ref, dis_ref, b_ref, w_ref, out_ref):
    agg = p_ref[0] + p_ref[1] + hs_ref[...]
    h = jnp.maximum(agg * dis_ref[...] + b_ref[...], 0.0)
    out_ref[...] = jnp.dot(
        h, w_ref[...], preferred_element_type=jnp.float32) * dis_ref[...]

  return pl.pallas_call(
      body,
      grid=(NBLK,),
      in_specs=[
          pl.BlockSpec((2, 128, D), lambda i: (0, i, 0)),
          pl.BlockSpec((128, D), lambda i: (i, 0)),
          pl.BlockSpec((128, 1), lambda i: (i, 0)),
          pl.BlockSpec((1, D), lambda i: (0, 0)),
          pl.BlockSpec((D, D), lambda i: (0, 0)),
      ],
      out_specs=pl.BlockSpec((128, D), lambda i: (i, 0)),
      out_shape=jax.ShapeDtypeStruct((NPAD, D), jnp.float32),
  )(parts, hs1, dis, b1, w2)


def _tc_final(parts, hs2, dis, b2):
  def body(p_ref, hs_ref, dis_ref, b_ref, out_ref):
    agg = p_ref[0] + p_ref[1] + hs_ref[...]
    out_ref[...] = agg * dis_ref[...] + b_ref[...]

  return pl.pallas_call(
      body,
      grid=(NBLK,),
      in_specs=[
          pl.BlockSpec((2, 128, D), lambda i: (0, i, 0)),
          pl.BlockSpec((128, D), lambda i: (i, 0)),
          pl.BlockSpec((128, 1), lambda i: (i, 0)),
          pl.BlockSpec((1, D), lambda i: (0, 0)),
      ],
      out_specs=pl.BlockSpec((128, D), lambda i: (i, 0)),
      out_shape=jax.ShapeDtypeStruct((NPAD, D), jnp.float32),
  )(parts, hs2, dis, b2)


def kernel(x, edge_index, W1, b1, W2, b2):
  src = edge_index[0].astype(jnp.int32)
  dst = edge_index[1].astype(jnp.int32)
  npad_e = EPAD - src.shape[0]
  src_flat = jnp.concatenate([src, jnp.zeros((npad_e,), jnp.int32)])
  dst_flat = jnp.concatenate([dst, jnp.full((npad_e,), N_NODES, jnp.int32)])
  x_pad = jnp.pad(x, ((0, NPAD - N_NODES), (0, 0)))
  b1r = b1.reshape(1, D)
  b2r = b2.reshape(1, D)
  zer = jnp.zeros((B, D), jnp.float32)
  one = jnp.ones((B, D), jnp.float32)

  degp = _sc_degree(dst_flat, zer, one).reshape(2, NPAD, D)
  hs1, dis = _tc_prescale(x_pad, degp, W1)
  parts1 = _sc_aggregate(hs1, src_flat, dst_flat, zer).reshape(2, NPAD, D)
  hs2 = _tc_mid(parts1, hs1, dis, b1r, W2)
  parts2 = _sc_aggregate(hs2, src_flat, dst_flat, zer).reshape(2, NPAD, D)
  out = _tc_final(parts2, hs2, dis, b2r)
  return out[:N_NODES]


# speedup vs baseline: 1.1088x; 1.0036x over previous
import functools

import jax
import jax.numpy as jnp
from jax import lax
from jax.experimental import pallas as pl
from jax.experimental.pallas import tpu as pltpu
from jax.experimental.pallas import tpu_sc as plsc

N_NODES = 10000
D = 128

NPAD = 10240
NBLK = NPAD // 128
NW = 32
B = 128
NCH = 80
EW = NCH * B
EPAD = NW * EW
ROWS_PER_TILE = NPAD // 16
NT = ROWS_PER_TILE // B

BA = 64
NCHA = EW // BA
NTA = ROWS_PER_TILE // BA

_MESH = dict(core_axis_name="c", subcore_axis_name="s")


def _sc_degree(dst_flat, zer, one):
  @functools.partial(
      pl.kernel,
      mesh=plsc.VectorSubcoreMesh(**_MESH),
      out_type=jax.ShapeDtypeStruct((2 * NPAD, D), jnp.float32),
      scratch_types=(
          [pltpu.VMEM((B,), jnp.int32)] * 4 +
          [pltpu.VMEM((B, D), jnp.float32)] * 2 +
          [pltpu.VMEM_SHARED((NPAD, D), jnp.float32)] +
          [pltpu.SemaphoreType.DMA] * 8
      ),
  )
  def k(dst_hbm, zer_hbm, one_hbm, out_hbm, *sc):
    didx = sc[0:4]
    ones_v, zbuf = sc[4:6]
    acc = sc[6]
    jsem = sc[7:11]
    tsem = sc[11:13]
    wsem = sc[13:15]
    c = lax.axis_index("c")
    s = lax.axis_index("s")
    w = c * 16 + s
    base = w * EW

    for q in range(4):
      pltpu.async_copy(dst_hbm.at[pl.ds(base + q * B, B)], didx[q], jsem[q])

    pltpu.sync_copy(zer_hbm, zbuf)
    for t in range(NT):
      pltpu.sync_copy(zbuf, acc.at[pl.ds(s * ROWS_PER_TILE + t * B, B)])
    pltpu.sync_copy(one_hbm, ones_v)
    plsc.subcore_barrier()

    for b in range(2):
      pltpu.make_async_copy(
          dst_hbm.at[pl.ds(base + b * B, B)], didx[b], jsem[b]).wait()
      pltpu.async_copy(ones_v, acc.at[didx[b]], tsem[b], add=True)

    def body(g, _):
      j0 = 4 * g + 2
      for p in range(4):
        u = (p + 2) % 4
        b = p % 2
        u2 = p
        pltpu.make_async_copy(ones_v, acc.at[didx[u2]], tsem[b]).wait()
        pltpu.async_copy(
            dst_hbm.at[pl.ds(base + (j0 + p + 2) * B, B)], didx[u2], jsem[u2])
        pltpu.make_async_copy(
            dst_hbm.at[pl.ds(base + (j0 + p) * B, B)], didx[u], jsem[u]).wait()
        pltpu.async_copy(ones_v, acc.at[didx[u]], tsem[b], add=True)
      return 0
    lax.fori_loop(0, (NCH - 4) // 4, body, 0)

    for p in range(2):
      u = 2 + p
      pltpu.make_async_copy(ones_v, acc.at[didx[p]], tsem[p]).wait()
      pltpu.make_async_copy(
          dst_hbm.at[pl.ds(base + (NCH - 2 + p) * B, B)], didx[u], jsem[u]).wait()
      pltpu.async_copy(ones_v, acc.at[didx[u]], tsem[p], add=True)
    for p in range(2):
      pltpu.make_async_copy(ones_v, acc.at[didx[2 + p]], tsem[p]).wait()

    plsc.subcore_barrier()
    wb = (ones_v, zbuf)
    for t in range(NT):
      b = t % 2
      if t >= 2:
        rp = s * ROWS_PER_TILE + (t - 2) * B
        pltpu.make_async_copy(
            wb[b], out_hbm.at[pl.ds(c * NPAD + rp, B)], wsem[b]).wait()
      r = s * ROWS_PER_TILE + t * B
      pltpu.sync_copy(acc.at[pl.ds(r, B)], wb[b])
      pltpu.async_copy(wb[b], out_hbm.at[pl.ds(c * NPAD + r, B)], wsem[b])
    for t in range(NT - 2, NT):
      b = t % 2
      r = s * ROWS_PER_TILE + t * B
      pltpu.make_async_copy(
          wb[b], out_hbm.at[pl.ds(c * NPAD + r, B)], wsem[b]).wait()

  return k(dst_flat, zer, one)


def _sc_aggregate(hs, src_flat, dst_flat, zer):
  @functools.partial(
      pl.kernel,
      mesh=plsc.VectorSubcoreMesh(**_MESH),
      out_type=jax.ShapeDtypeStruct((2 * NPAD, D), jnp.float32),
      scratch_types=(
          [pltpu.VMEM((BA,), jnp.int32)] * 8 +
          [pltpu.VMEM((BA,), jnp.int32)] * 8 +
          [pltpu.VMEM((BA, D), jnp.float32)] * 4 +
          [pltpu.VMEM_SHARED((NPAD, D), jnp.float32)] +
          [pltpu.SemaphoreType.DMA] * 24
      ),
  )
  def k(hs_hbm, src_hbm, dst_hbm, zer_hbm, out_hbm, *sc):
    sidx = sc[0:8]
    didx = sc[8:16]
    buf = sc[16:20]
    acc = sc[20]
    isem = sc[21:29]
    jsem = sc[29:37]
    gsem = sc[37:41]
    tsem = sc[41:45]
    c = lax.axis_index("c")
    s = lax.axis_index("s")
    w = c * 16 + s
    base = w * EW

    for q in range(8):
      pltpu.async_copy(src_hbm.at[pl.ds(base + q * BA, BA)], sidx[q], isem[q])
      pltpu.async_copy(dst_hbm.at[pl.ds(base + q * BA, BA)], didx[q], jsem[q])

    pltpu.sync_copy(zer_hbm, buf[0])
    for t in range(NTA):
      pltpu.sync_copy(buf[0], acc.at[pl.ds(s * ROWS_PER_TILE + t * BA, BA)])

    for b in range(4):
      pltpu.make_async_copy(
          src_hbm.at[pl.ds(base + b * BA, BA)], sidx[b], isem[b]).wait()
      pltpu.make_async_copy(
          dst_hbm.at[pl.ds(base + b * BA, BA)], didx[b], jsem[b]).wait()
      pltpu.async_copy(hs_hbm.at[sidx[b]], buf[b], gsem[b])
    plsc.subcore_barrier()

    def body(g, _):
      j0 = 8 * g
      for p in range(2):
        for b in range(4):
          u = (4 * p + b) % 8
          pltpu.make_async_copy(hs_hbm.at[sidx[u]], buf[b], gsem[b]).wait()
          pltpu.async_copy(buf[b], acc.at[didx[u]], tsem[b], add=True)
        for b in range(4):
          u = (4 * p + b) % 8
          u2 = (u + 4) % 8
          off4 = base + (j0 + 4 * p + b + 4) * BA
          off8 = base + (j0 + 4 * p + b + 8) * BA
          pltpu.make_async_copy(buf[b], acc.at[didx[u]], tsem[b]).wait()
          pltpu.make_async_copy(
              src_hbm.at[pl.ds(off4, BA)], sidx[u2], isem[u2]).wait()
          pltpu.make_async_copy(
              dst_hbm.at[pl.ds(off4, BA)], didx[u2], jsem[u2]).wait()
          pltpu.async_copy(hs_hbm.at[sidx[u2]], buf[b], gsem[b])
          pltpu.async_copy(src_hbm.at[pl.ds(off8, BA)], sidx[u], isem[u])
          pltpu.async_copy(dst_hbm.at[pl.ds(off8, BA)], didx[u], jsem[u])
      return 0
    lax.fori_loop(0, (NCHA - 8) // 8, body, 0)

    for b in range(4):
      pltpu.make_async_copy(hs_hbm.at[sidx[b]], buf[b], gsem[b]).wait()
      pltpu.async_copy(buf[b], acc.at[didx[b]], tsem[b], add=True)
    for b in range(4):
      u2 = b + 4
      off4 = base + (NCHA - 4 + b) * BA
      pltpu.make_async_copy(buf[b], acc.at[didx[b]], tsem[b]).wait()
      pltpu.make_async_copy(
          src_hbm.at[pl.ds(off4, BA)], sidx[u2], isem[u2]).wait()
      pltpu.make_async_copy(
          dst_hbm.at[pl.ds(off4, BA)], didx[u2], jsem[u2]).wait()
      pltpu.async_copy(hs_hbm.at[sidx[u2]], buf[b], gsem[b])
    for b in range(4):
      u = b + 4
      pltpu.make_async_copy(hs_hbm.at[sidx[u]], buf[b], gsem[b]).wait()
      pltpu.async_copy(buf[b], acc.at[didx[u]], tsem[b], add=True)
    for b in range(4):
      u = b + 4
      pltpu.make_async_copy(buf[b], acc.at[didx[u]], tsem[b]).wait()

    plsc.subcore_barrier()
    for t in range(NTA):
      b = t % 4
      if t >= 4:
        rp = s * ROWS_PER_TILE + (t - 4) * BA
        pltpu.make_async_copy(
            buf[b], out_hbm.at[pl.ds(c * NPAD + rp, BA)], gsem[b]).wait()
      r = s * ROWS_PER_TILE + t * BA
      pltpu.sync_copy(acc.at[pl.ds(r, BA)], buf[b])
      pltpu.async_copy(buf[b], out_hbm.at[pl.ds(c * NPAD + r, BA)], gsem[b])
    for t in range(NTA - 4, NTA):
      b = t % 4
      r = s * ROWS_PER_TILE + t * BA
      pltpu.make_async_copy(
          buf[b], out_hbm.at[pl.ds(c * NPAD + r, BA)], gsem[b]).wait()

  return k(hs, src_flat, dst_flat, zer)


def _tc_prescale(x_pad, degp, w1):
  def body(x_ref, dp_ref, w_ref, hs_ref, dis_ref):
    deg = dp_ref[0, :, 0] + dp_ref[1, :, 0] + 1.0
    dis = lax.rsqrt(deg)
    h = jnp.dot(x_ref[...], w_ref[...], preferred_element_type=jnp.float32)
    hs_ref[...] = h * dis[:, None]
    dis_ref[...] = dis[:, None]

  return pl.pallas_call(
      body,
      grid=(NBLK,),
      in_specs=[
          pl.BlockSpec((128, D), lambda i: (i, 0)),
          pl.BlockSpec((2, 128, D), lambda i: (0, i, 0)),
          pl.BlockSpec((D, D), lambda i: (0, 0)),
      ],
      out_specs=[
          pl.BlockSpec((128, D), lambda i: (i, 0)),
          pl.BlockSpec((128, 1), lambda i: (i, 0)),
      ],
      out_shape=[
          jax.ShapeDtypeStruct((NPAD, D), jnp.float32),
          jax.ShapeDtypeStruct((NPAD, 1), jnp.float32),
      ],
  )(x_pad, degp, w1)


def _tc_mid(parts, hs1, dis, b1, w2):
  def body(p_ref, hs_ref, dis_ref, b_ref, w_ref, out_ref):
    agg = p_ref[0] + p_ref[1] + hs_ref[...]
    h = jnp.maximum(agg * dis_ref[...] + b_ref[...], 0.0)
    out_ref[...] = jnp.dot(
        h, w_ref[...], preferred_element_type=jnp.float32) * dis_ref[...]

  return pl.pallas_call(
      body,
      grid=(NBLK,),
      in_specs=[
          pl.BlockSpec((2, 128, D), lambda i: (0, i, 0)),
          pl.BlockSpec((128, D), lambda i: (i, 0)),
          pl.BlockSpec((128, 1), lambda i: (i, 0)),
          pl.BlockSpec((1, D), lambda i: (0, 0)),
          pl.BlockSpec((D, D), lambda i: (0, 0)),
      ],
      out_specs=pl.BlockSpec((128, D), lambda i: (i, 0)),
      out_shape=jax.ShapeDtypeStruct((NPAD, D), jnp.float32),
  )(parts, hs1, dis, b1, w2)


def _tc_final(parts, hs2, dis, b2):
  def body(p_ref, hs_ref, dis_ref, b_ref, out_ref):
    agg = p_ref[0] + p_ref[1] + hs_ref[...]
    out_ref[...] = agg * dis_ref[...] + b_ref[...]

  return pl.pallas_call(
      body,
      grid=(NBLK,),
      in_specs=[
          pl.BlockSpec((2, 128, D), lambda i: (0, i, 0)),
          pl.BlockSpec((128, D), lambda i: (i, 0)),
          pl.BlockSpec((128, 1), lambda i: (i, 0)),
          pl.BlockSpec((1, D), lambda i: (0, 0)),
      ],
      out_specs=pl.BlockSpec((128, D), lambda i: (i, 0)),
      out_shape=jax.ShapeDtypeStruct((NPAD, D), jnp.float32),
  )(parts, hs2, dis, b2)


def kernel(x, edge_index, W1, b1, W2, b2):
  src = edge_index[0].astype(jnp.int32)
  dst = edge_index[1].astype(jnp.int32)
  npad_e = EPAD - src.shape[0]
  src_flat = jnp.concatenate([src, jnp.zeros((npad_e,), jnp.int32)])
  dst_flat = jnp.concatenate([dst, jnp.full((npad_e,), N_NODES, jnp.int32)])
  x_pad = jnp.pad(x, ((0, NPAD - N_NODES), (0, 0)))
  b1r = b1.reshape(1, D)
  b2r = b2.reshape(1, D)
  zer = jnp.zeros((B, D), jnp.float32)
  zera = jnp.zeros((BA, D), jnp.float32)
  one = jnp.ones((B, D), jnp.float32)

  degp = _sc_degree(dst_flat, zer, one).reshape(2, NPAD, D)
  hs1, dis = _tc_prescale(x_pad, degp, W1)
  parts1 = _sc_aggregate(hs1, src_flat, dst_flat, zera).reshape(2, NPAD, D)
  hs2 = _tc_mid(parts1, hs1, dis, b1r, W2)
  parts2 = _sc_aggregate(hs2, src_flat, dst_flat, zera).reshape(2, NPAD, D)
  out = _tc_final(parts2, hs2, dis, b2r)
  return out[:N_NODES]
